# baseline (speedup 1.0000x reference)
import functools

import jax
import jax.numpy as jnp
from jax import lax
from jax.experimental import pallas as pl
from jax.experimental.pallas import tpu as pltpu
from jax.experimental.pallas import tpu_sc as plsc

D_MODEL = 1024
D_FF = 2048
E = 64
TOP_K = 2
S = 2048

BLK = 128
NPAIR = S * TOP_K
NB = NPAIR // BLK + E
NBP = 128
P_PAD = NB * BLK
TB = 512
NTB = S // TB

_NW = 32
_TPW = S // _NW
_CCH = 16
_NCC = _TPW // _CCH


def _router_body(x_ref, wg_ref, g1_ref, g2_ref, d1_ref, d2_ref,
                 b2e_ref, nact_ref, xb_ref, base_ref, i1s, i2s, r1s, r2s):
    pid = pl.program_id(0)

    @pl.when(pid == 0)
    def _():
        base_ref[...] = jnp.zeros((1, E), jnp.float32)

    xb_ref[...] = x_ref[...].astype(jnp.bfloat16)

    logits = jnp.dot(x_ref[...], wg_ref[...], preferred_element_type=jnp.float32)
    iota = lax.broadcasted_iota(jnp.int32, logits.shape, 1)
    m1 = jnp.max(logits, axis=1, keepdims=True)
    i1 = jnp.min(jnp.where(logits == m1, iota, E), axis=1, keepdims=True)
    masked = jnp.where(iota == i1, -jnp.inf, logits)
    m2 = jnp.max(masked, axis=1, keepdims=True)
    i2 = jnp.min(jnp.where(masked == m2, iota, E), axis=1, keepdims=True)
    g1 = 1.0 / (1.0 + jnp.exp(m2 - m1))
    g1_ref[...] = jnp.broadcast_to(g1, (TB, 16))
    g2_ref[...] = jnp.broadcast_to(1.0 - g1, (TB, 16))

    oh1 = (iota == i1).astype(jnp.float32)
    oh2 = (iota == i2).astype(jnp.float32)
    ri = lax.broadcasted_iota(jnp.int32, (TB, TB), 0)
    ci = lax.broadcasted_iota(jnp.int32, (TB, TB), 1)
    ltri = (ri > ci).astype(jnp.float32)
    cum = jnp.dot(ltri, oh1 + oh2, preferred_element_type=jnp.float32)
    base = base_ref[...]
    r1 = jnp.sum(oh1 * (base + cum), axis=1, keepdims=True)
    r2 = jnp.sum(oh2 * (base + cum + oh1), axis=1, keepdims=True)
    sl = pl.ds(pid * TB, TB)
    i1s[sl, :] = i1
    i2s[sl, :] = i2
    r1s[sl, :] = r1
    r2s[sl, :] = r2
    newbase = base + jnp.sum(oh1 + oh2, axis=0, keepdims=True)
    base_ref[...] = newbase

    @pl.when(pid == NTB - 1)
    def _():
        counts = newbase
        nblk = jnp.floor((counts + (BLK - 1)) * (1.0 / BLK))
        ei = lax.broadcasted_iota(jnp.int32, (E, E), 0)
        ej = lax.broadcasted_iota(jnp.int32, (E, E), 1)
        incl = (ei <= ej).astype(jnp.float32)
        cum_incl = jnp.dot(nblk, incl, preferred_element_type=jnp.float32)
        blk_start = cum_incl - nblk
        row_off = blk_start * float(BLK)

        it = lax.broadcasted_iota(jnp.int32, (S, E), 1)
        sel1 = (it == i1s[...]).astype(jnp.float32)
        sel2 = (it == i2s[...]).astype(jnp.float32)
        d1 = jnp.sum(sel1 * row_off, axis=1, keepdims=True) + r1s[...]
        d2 = jnp.sum(sel2 * row_off, axis=1, keepdims=True) + r2s[...]
        d1_ref[...] = d1.astype(jnp.int32)
        d2_ref[...] = d2.astype(jnp.int32)

        bi = lax.broadcasted_iota(jnp.int32, (NBP, E), 0)
        be = lax.broadcasted_iota(jnp.int32, (NBP, E), 1)
        active = (bi >= blk_start.astype(jnp.int32)) & (nblk > 0.0)
        b2e_ref[...] = jnp.max(jnp.where(active, be, 0), axis=1, keepdims=True)
        nact_ref[...] = jnp.sum(nblk, axis=1, keepdims=True).astype(jnp.int32)


def _router(x, wg):
    outs = [
        jax.ShapeDtypeStruct((S, 16), jnp.float32),
        jax.ShapeDtypeStruct((S, 16), jnp.float32),
        jax.ShapeDtypeStruct((S, 1), jnp.int32),
        jax.ShapeDtypeStruct((S, 1), jnp.int32),
        jax.ShapeDtypeStruct((NBP, 1), jnp.int32),
        jax.ShapeDtypeStruct((1, 1), jnp.int32),
        jax.ShapeDtypeStruct((S, D_MODEL), jnp.bfloat16),
    ]
    tokspec = pl.BlockSpec((TB, 16), lambda i: (i, 0))
    whole = lambda i: (0, 0)
    return pl.pallas_call(
        _router_body,
        grid=(NTB,),
        in_specs=[
            pl.BlockSpec((TB, D_MODEL), lambda i: (i, 0)),
            pl.BlockSpec((D_MODEL, E), whole),
        ],
        out_specs=[tokspec, tokspec,
                   pl.BlockSpec((S, 1), whole), pl.BlockSpec((S, 1), whole),
                   pl.BlockSpec((NBP, 1), whole), pl.BlockSpec((1, 1), whole),
                   pl.BlockSpec((TB, D_MODEL), lambda i: (i, 0))],
        out_shape=outs,
        scratch_shapes=[pltpu.VMEM((1, E), jnp.float32),
                        pltpu.VMEM((S, 1), jnp.int32),
                        pltpu.VMEM((S, 1), jnp.int32),
                        pltpu.VMEM((S, 1), jnp.float32),
                        pltpu.VMEM((S, 1), jnp.float32)],
    )(x, wg)


def _sc_dispatch(x2d, d1w, d2w):
    mesh = plsc.VectorSubcoreMesh(core_axis_name="c", subcore_axis_name="s")

    @functools.partial(
        pl.kernel,
        mesh=mesh,
        out_type=jax.ShapeDtypeStruct((P_PAD, D_MODEL // 2), jnp.int32),
        scratch_types=[
            pltpu.VMEM((TOP_K, _TPW), jnp.int32),
            pltpu.VMEM((_TPW, D_MODEL // 2), jnp.int32),
            pltpu.SemaphoreType.DMA,
        ],
    )
    def k(x_hbm, d1_hbm, d2_hbm, out_hbm, idx_v, buf, sem):
        wid = lax.axis_index("s") * 2 + lax.axis_index("c")
        pltpu.sync_copy(d1_hbm.at[wid], idx_v.at[0])
        pltpu.sync_copy(d2_hbm.at[wid], idx_v.at[1])
        pltpu.sync_copy(x_hbm.at[pl.ds(wid * _TPW, _TPW)], buf)
        ca = pltpu.async_copy(buf, out_hbm.at[idx_v.at[0]], sem)
        cb = pltpu.async_copy(buf, out_hbm.at[idx_v.at[1]], sem)
        ca.wait()
        cb.wait()

    return k(x2d, d1w, d2w)


def _ffn_body(b2e_ref, nact_ref, x_ref, w1_ref, b1_ref, w2_ref, b2_ref, y_ref):
    pid = pl.program_id(0)

    @pl.when(pid < nact_ref[0])
    def _():
        x = x_ref[...].astype(jnp.float32)
        h = jnp.dot(x, w1_ref[0], preferred_element_type=jnp.float32)
        h = jax.nn.gelu(h + b1_ref[0])
        y = jnp.dot(h, w2_ref[0], preferred_element_type=jnp.float32)
        y_ref[...] = y + b2_ref[0]


def _ffn(x_pad, w1, b1, w2, b2, b2e, nact):
    clamp = lambda i, na: jnp.where(i < na[0], i, na[0] - 1)
    grid_spec = pltpu.PrefetchScalarGridSpec(
        num_scalar_prefetch=2,
        grid=(NB,),
        in_specs=[
            pl.BlockSpec((BLK, D_MODEL),
                         lambda i, b2e, na: (clamp(i, na), 0)),
            pl.BlockSpec((1, D_MODEL, D_FF), lambda i, b2e, na: (b2e[i], 0, 0)),
            pl.BlockSpec((1, 1, D_FF), lambda i, b2e, na: (b2e[i], 0, 0)),
            pl.BlockSpec((1, D_FF, D_MODEL), lambda i, b2e, na: (b2e[i], 0, 0)),
            pl.BlockSpec((1, 1, D_MODEL), lambda i, b2e, na: (b2e[i], 0, 0)),
        ],
        out_specs=pl.BlockSpec((BLK, D_MODEL),
                               lambda i, b2e, na: (clamp(i, na), 0)),
    )
    return pl.pallas_call(
        _ffn_body,
        grid_spec=grid_spec,
        out_shape=jax.ShapeDtypeStruct((P_PAD, D_MODEL), jnp.float32),
    )(b2e, nact, x_pad, w1, b1, w2, b2)


def _sc_combine(y_pad, d1w, d2w, g1w, g2w):
    mesh = plsc.VectorSubcoreMesh(core_axis_name="c", subcore_axis_name="s")

    @functools.partial(
        pl.kernel,
        mesh=mesh,
        out_type=jax.ShapeDtypeStruct((S, D_MODEL), jnp.float32),
        scratch_types=[
            pltpu.VMEM((_TPW,), jnp.int32),
            pltpu.VMEM((_TPW,), jnp.int32),
            pltpu.VMEM((_TPW, 16), jnp.float32),
            pltpu.VMEM((_TPW, 16), jnp.float32),
            pltpu.VMEM((2, _CCH, D_MODEL), jnp.float32),
            pltpu.VMEM((2, _CCH, D_MODEL), jnp.float32),
            pltpu.VMEM((2, _CCH, D_MODEL), jnp.float32),
            pltpu.SemaphoreType.DMA,
            pltpu.SemaphoreType.DMA,
            pltpu.SemaphoreType.DMA,
            pltpu.SemaphoreType.DMA,
            pltpu.SemaphoreType.DMA,
            pltpu.SemaphoreType.DMA,
        ],
    )
    def k(y_hbm, d1_hbm, d2_hbm, g1_hbm, g2_hbm, out_hbm,
          pa_v, pb_v, ga_v, gb_v, bufa, bufb, bufo,
          sga0, sga1, sgb0, sgb1, swr0, swr1):
        sga = [sga0, sga1]
        sgb = [sgb0, sgb1]
        swr = [swr0, swr1]
        wid = lax.axis_index("s") * 2 + lax.axis_index("c")
        pltpu.sync_copy(d1_hbm.at[wid], pa_v)
        pltpu.sync_copy(d2_hbm.at[wid], pb_v)
        pltpu.sync_copy(g1_hbm.at[wid], ga_v)
        pltpu.sync_copy(g2_hbm.at[wid], gb_v)

        def gathers(c, s):
            cpa = pltpu.async_copy(
                y_hbm.at[pa_v.at[pl.ds(c * _CCH, _CCH)]], bufa.at[s], sga[s])
            cpb = pltpu.async_copy(
                y_hbm.at[pb_v.at[pl.ds(c * _CCH, _CCH)]], bufb.at[s], sgb[s])
            return cpa, cpb

        pend = [gathers(0, 0), gathers(1, 1)]
        wr = [None, None]
        for c in range(_NCC):
            s = c % 2
            cpa, cpb = pend[s]
            cpa.wait()
            cpb.wait()
            if wr[s] is not None:
                wr[s].wait()

            def body(r, carry, c=c, s=s):
                tok = c * _CCH + r
                ga = ga_v[tok, :]
                gb = gb_v[tok, :]
                for q in range(D_MODEL // 16):
                    sl = pl.ds(q * 16, 16)
                    bufo[s, r, sl] = ga * bufa[s, r, sl] + gb * bufb[s, r, sl]
                return carry

            lax.fori_loop(0, _CCH, body, 0)
            if c + 2 < _NCC:
                pend[s] = gathers(c + 2, s)
            wr[s] = pltpu.async_copy(
                bufo.at[s], out_hbm.at[pl.ds(wid * _TPW + c * _CCH, _CCH)],
                swr[s])
        for s in range(2):
            if wr[s] is not None:
                wr[s].wait()

    return k(y_pad, d1w, d2w, g1w, g2w)


def kernel(hidden_states, Wg, W1, b1, W2, b2):
    x = hidden_states.reshape(S, D_MODEL)
    g1, g2, dest1, dest2, b2e, nact, xb = _router(x, Wg)

    d1w = dest1.reshape(_NW, _TPW)
    d2w = dest2.reshape(_NW, _TPW)
    xb32 = jax.lax.bitcast_convert_type(
        xb.reshape(S, D_MODEL // 2, 2), jnp.int32)
    xp32 = _sc_dispatch(xb32, d1w, d2w)
    x_pad = jax.lax.bitcast_convert_type(
        xp32, jnp.bfloat16).reshape(P_PAD, D_MODEL)
    y_pad = _ffn(x_pad, W1, b1.reshape(E, 1, D_FF), W2,
                 b2.reshape(E, 1, D_MODEL), b2e.reshape(NBP), nact.reshape(1))

    out = _sc_combine(y_pad, d1w, d2w,
                      g1.reshape(_NW, _TPW, 16), g2.reshape(_NW, _TPW, 16))
    return out.reshape(hidden_states.shape)


# speedup vs baseline: 1.7578x; 1.7578x over previous
import functools

import jax
import jax.numpy as jnp
from jax import lax
from jax.experimental import pallas as pl
from jax.experimental.pallas import tpu as pltpu
from jax.experimental.pallas import tpu_sc as plsc

D_MODEL = 1024
D_FF = 2048
E = 64
TOP_K = 2
S = 2048

BLK = 128
NPAIR = S * TOP_K
NB = NPAIR // BLK + E
NBP = 128
P_PAD = NB * BLK
TB = 512
NTB = S // TB

_NW = 32
_TPW = S // _NW
_CCH = 16
_NCC = _TPW // _CCH


def _router_body(x_ref, wg_ref, g1_ref, g2_ref, d1_ref, d2_ref,
                 b2e_ref, nact_ref, xb_ref, base_ref, i1s, i2s, r1s, r2s):
    pid = pl.program_id(0)

    @pl.when(pid == 0)
    def _():
        base_ref[...] = jnp.zeros((1, E), jnp.float32)

    xu = lax.bitcast_convert_type(x_ref[...], jnp.int32)
    rnd = 0x7FFF + ((xu >> 16) & 1)
    xb16 = ((xu + rnd) >> 16) & 0xFFFF
    lo = xb16[:, : D_MODEL // 2]
    hi = xb16[:, D_MODEL // 2:]
    xb_ref[...] = (hi << 16) | lo

    logits = jnp.dot(x_ref[...], wg_ref[...], preferred_element_type=jnp.float32)
    iota = lax.broadcasted_iota(jnp.int32, logits.shape, 1)
    m1 = jnp.max(logits, axis=1, keepdims=True)
    i1 = jnp.min(jnp.where(logits == m1, iota, E), axis=1, keepdims=True)
    masked = jnp.where(iota == i1, -jnp.inf, logits)
    m2 = jnp.max(masked, axis=1, keepdims=True)
    i2 = jnp.min(jnp.where(masked == m2, iota, E), axis=1, keepdims=True)
    g1 = 1.0 / (1.0 + jnp.exp(m2 - m1))
    g1_ref[...] = jnp.broadcast_to(g1, (TB, 16))
    g2_ref[...] = jnp.broadcast_to(1.0 - g1, (TB, 16))

    oh1 = (iota == i1).astype(jnp.float32)
    oh2 = (iota == i2).astype(jnp.float32)
    ri = lax.broadcasted_iota(jnp.int32, (TB, TB), 0)
    ci = lax.broadcasted_iota(jnp.int32, (TB, TB), 1)
    ltri = (ri > ci).astype(jnp.float32)
    cum = jnp.dot(ltri, oh1 + oh2, preferred_element_type=jnp.float32)
    base = base_ref[...]
    r1 = jnp.sum(oh1 * (base + cum), axis=1, keepdims=True)
    r2 = jnp.sum(oh2 * (base + cum + oh1), axis=1, keepdims=True)
    sl = pl.ds(pid * TB, TB)
    i1s[sl, :] = i1
    i2s[sl, :] = i2
    r1s[sl, :] = r1
    r2s[sl, :] = r2
    newbase = base + jnp.sum(oh1 + oh2, axis=0, keepdims=True)
    base_ref[...] = newbase

    @pl.when(pid == NTB - 1)
    def _():
        counts = newbase
        nblk = jnp.floor((counts + (BLK - 1)) * (1.0 / BLK))
        ei = lax.broadcasted_iota(jnp.int32, (E, E), 0)
        ej = lax.broadcasted_iota(jnp.int32, (E, E), 1)
        incl = (ei <= ej).astype(jnp.float32)
        cum_incl = jnp.dot(nblk, incl, preferred_element_type=jnp.float32)
        blk_start = cum_incl - nblk
        row_off = blk_start * float(BLK)

        it = lax.broadcasted_iota(jnp.int32, (S, E), 1)
        sel1 = (it == i1s[...]).astype(jnp.float32)
        sel2 = (it == i2s[...]).astype(jnp.float32)
        d1 = jnp.sum(sel1 * row_off, axis=1, keepdims=True) + r1s[...]
        d2 = jnp.sum(sel2 * row_off, axis=1, keepdims=True) + r2s[...]
        d1_ref[...] = d1.astype(jnp.int32)
        d2_ref[...] = d2.astype(jnp.int32)

        bi = lax.broadcasted_iota(jnp.int32, (NBP, E), 0)
        be = lax.broadcasted_iota(jnp.int32, (NBP, E), 1)
        active = (bi >= blk_start.astype(jnp.int32)) & (nblk > 0.0)
        b2e_ref[...] = jnp.max(jnp.where(active, be, 0), axis=1, keepdims=True)
        nact_ref[...] = jnp.sum(nblk, axis=1, keepdims=True).astype(jnp.int32)


def _router(x, wg):
    outs = [
        jax.ShapeDtypeStruct((S, 16), jnp.float32),
        jax.ShapeDtypeStruct((S, 16), jnp.float32),
        jax.ShapeDtypeStruct((S, 1), jnp.int32),
        jax.ShapeDtypeStruct((S, 1), jnp.int32),
        jax.ShapeDtypeStruct((NBP, 1), jnp.int32),
        jax.ShapeDtypeStruct((1, 1), jnp.int32),
        jax.ShapeDtypeStruct((S, D_MODEL // 2), jnp.int32),
    ]
    tokspec = pl.BlockSpec((TB, 16), lambda i: (i, 0))
    whole = lambda i: (0, 0)
    return pl.pallas_call(
        _router_body,
        grid=(NTB,),
        in_specs=[
            pl.BlockSpec((TB, D_MODEL), lambda i: (i, 0)),
            pl.BlockSpec((D_MODEL, E), whole),
        ],
        out_specs=[tokspec, tokspec,
                   pl.BlockSpec((S, 1), whole), pl.BlockSpec((S, 1), whole),
                   pl.BlockSpec((NBP, 1), whole), pl.BlockSpec((1, 1), whole),
                   pl.BlockSpec((TB, D_MODEL // 2), lambda i: (i, 0))],
        out_shape=outs,
        scratch_shapes=[pltpu.VMEM((1, E), jnp.float32),
                        pltpu.VMEM((S, 1), jnp.int32),
                        pltpu.VMEM((S, 1), jnp.int32),
                        pltpu.VMEM((S, 1), jnp.float32),
                        pltpu.VMEM((S, 1), jnp.float32)],
    )(x, wg)


def _sc_dispatch(x2d, d1w, d2w):
    mesh = plsc.VectorSubcoreMesh(core_axis_name="c", subcore_axis_name="s")

    @functools.partial(
        pl.kernel,
        mesh=mesh,
        out_type=jax.ShapeDtypeStruct((P_PAD, D_MODEL // 2), jnp.int32),
        scratch_types=[
            pltpu.VMEM((TOP_K, _TPW), jnp.int32),
            pltpu.VMEM((_TPW, D_MODEL // 2), jnp.int32),
            pltpu.SemaphoreType.DMA,
        ],
    )
    def k(x_hbm, d1_hbm, d2_hbm, out_hbm, idx_v, buf, sem):
        wid = lax.axis_index("s") * 2 + lax.axis_index("c")
        pltpu.sync_copy(d1_hbm.at[wid], idx_v.at[0])
        pltpu.sync_copy(d2_hbm.at[wid], idx_v.at[1])
        pltpu.sync_copy(x_hbm.at[pl.ds(wid * _TPW, _TPW)], buf)
        ca = pltpu.async_copy(buf, out_hbm.at[idx_v.at[0]], sem)
        cb = pltpu.async_copy(buf, out_hbm.at[idx_v.at[1]], sem)
        ca.wait()
        cb.wait()

    return k(x2d, d1w, d2w)


def _ffn_body(b2e_ref, nact_ref, x_ref, w1_ref, b1_ref, w2_ref, b2_ref, y_ref):
    pid = pl.program_id(0)

    @pl.when(pid < nact_ref[0])
    def _():
        w = x_ref[...]
        lo = lax.bitcast_convert_type(w << 16, jnp.float32)
        hi = lax.bitcast_convert_type(w & jnp.int32(-65536), jnp.float32)
        x = jnp.concatenate([lo, hi], axis=1)
        h = jnp.dot(x, w1_ref[0], preferred_element_type=jnp.float32)
        h = jax.nn.gelu(h + b1_ref[0])
        y = jnp.dot(h, w2_ref[0], preferred_element_type=jnp.float32)
        y_ref[...] = y + b2_ref[0]


def _ffn(x_pad, w1, b1, w2, b2, b2e, nact):
    clamp = lambda i, na: jnp.where(i < na[0], i, na[0] - 1)
    grid_spec = pltpu.PrefetchScalarGridSpec(
        num_scalar_prefetch=2,
        grid=(NB,),
        in_specs=[
            pl.BlockSpec((BLK, D_MODEL // 2),
                         lambda i, b2e, na: (clamp(i, na), 0)),
            pl.BlockSpec((1, D_MODEL, D_FF), lambda i, b2e, na: (b2e[i], 0, 0)),
            pl.BlockSpec((1, 1, D_FF), lambda i, b2e, na: (b2e[i], 0, 0)),
            pl.BlockSpec((1, D_FF, D_MODEL), lambda i, b2e, na: (b2e[i], 0, 0)),
            pl.BlockSpec((1, 1, D_MODEL), lambda i, b2e, na: (b2e[i], 0, 0)),
        ],
        out_specs=pl.BlockSpec((BLK, D_MODEL),
                               lambda i, b2e, na: (clamp(i, na), 0)),
    )
    return pl.pallas_call(
        _ffn_body,
        grid_spec=grid_spec,
        out_shape=jax.ShapeDtypeStruct((P_PAD, D_MODEL), jnp.float32),
    )(b2e, nact, x_pad, w1, b1, w2, b2)


def _sc_combine(y_pad, d1w, d2w, g1w, g2w):
    mesh = plsc.VectorSubcoreMesh(core_axis_name="c", subcore_axis_name="s")

    @functools.partial(
        pl.kernel,
        mesh=mesh,
        out_type=jax.ShapeDtypeStruct((S, D_MODEL), jnp.float32),
        scratch_types=[
            pltpu.VMEM((_TPW,), jnp.int32),
            pltpu.VMEM((_TPW,), jnp.int32),
            pltpu.VMEM((_TPW, 16), jnp.float32),
            pltpu.VMEM((_TPW, 16), jnp.float32),
            pltpu.VMEM((2, _CCH, D_MODEL), jnp.float32),
            pltpu.VMEM((2, _CCH, D_MODEL), jnp.float32),
            pltpu.VMEM((2, _CCH, D_MODEL), jnp.float32),
            pltpu.SemaphoreType.DMA,
            pltpu.SemaphoreType.DMA,
            pltpu.SemaphoreType.DMA,
            pltpu.SemaphoreType.DMA,
            pltpu.SemaphoreType.DMA,
            pltpu.SemaphoreType.DMA,
        ],
    )
    def k(y_hbm, d1_hbm, d2_hbm, g1_hbm, g2_hbm, out_hbm,
          pa_v, pb_v, ga_v, gb_v, bufa, bufb, bufo,
          sga0, sga1, sgb0, sgb1, swr0, swr1):
        sga = [sga0, sga1]
        sgb = [sgb0, sgb1]
        swr = [swr0, swr1]
        wid = lax.axis_index("s") * 2 + lax.axis_index("c")
        pltpu.sync_copy(d1_hbm.at[wid], pa_v)
        pltpu.sync_copy(d2_hbm.at[wid], pb_v)
        pltpu.sync_copy(g1_hbm.at[wid], ga_v)
        pltpu.sync_copy(g2_hbm.at[wid], gb_v)

        def gathers(c, s):
            cpa = pltpu.async_copy(
                y_hbm.at[pa_v.at[pl.ds(c * _CCH, _CCH)]], bufa.at[s], sga[s])
            cpb = pltpu.async_copy(
                y_hbm.at[pb_v.at[pl.ds(c * _CCH, _CCH)]], bufb.at[s], sgb[s])
            return cpa, cpb

        pend = [gathers(0, 0), gathers(1, 1)]
        wr = [None, None]
        for c in range(_NCC):
            s = c % 2
            cpa, cpb = pend[s]
            cpa.wait()
            cpb.wait()
            if wr[s] is not None:
                wr[s].wait()

            def body(r, carry, c=c, s=s):
                tok = c * _CCH + r
                ga = ga_v[tok, :]
                gb = gb_v[tok, :]
                for q in range(D_MODEL // 16):
                    sl = pl.ds(q * 16, 16)
                    bufo[s, r, sl] = ga * bufa[s, r, sl] + gb * bufb[s, r, sl]
                return carry

            lax.fori_loop(0, _CCH, body, 0)
            if c + 2 < _NCC:
                pend[s] = gathers(c + 2, s)
            wr[s] = pltpu.async_copy(
                bufo.at[s], out_hbm.at[pl.ds(wid * _TPW + c * _CCH, _CCH)],
                swr[s])
        for s in range(2):
            if wr[s] is not None:
                wr[s].wait()

    return k(y_pad, d1w, d2w, g1w, g2w)


def kernel(hidden_states, Wg, W1, b1, W2, b2):
    x = hidden_states.reshape(S, D_MODEL)
    g1, g2, dest1, dest2, b2e, nact, xb = _router(x, Wg)

    d1w = dest1.reshape(_NW, _TPW)
    d2w = dest2.reshape(_NW, _TPW)
    x_pad = _sc_dispatch(xb, d1w, d2w)
    y_pad = _ffn(x_pad, W1, b1.reshape(E, 1, D_FF), W2,
                 b2.reshape(E, 1, D_MODEL), b2e.reshape(NBP), nact.reshape(1))

    out = _sc_combine(y_pad, d1w, d2w,
                      g1.reshape(_NW, _TPW, 16), g2.reshape(_NW, _TPW, 16))
    return out.reshape(hidden_states.shape)


# speedup vs baseline: 1.7646x; 1.0039x over previous
import functools

import jax
import jax.numpy as jnp
from jax import lax
from jax.experimental import pallas as pl
from jax.experimental.pallas import tpu as pltpu
from jax.experimental.pallas import tpu_sc as plsc

D_MODEL = 1024
D_FF = 2048
E = 64
TOP_K = 2
S = 2048

BLK = 128
NPAIR = S * TOP_K
NB = NPAIR // BLK + E
NBP = 128
P_PAD = NB * BLK
TB = 512
NTB = S // TB

_NW = 32
_TPW = S // _NW
_CCH = 16
_NCC = _TPW // _CCH


def _router_body(x_ref, wg_ref, g1_ref, g2_ref, d1_ref, d2_ref,
                 b2e_ref, nact_ref, xb_ref, base_ref, i1s, i2s, r1s, r2s):
    pid = pl.program_id(0)

    @pl.when(pid == 0)
    def _():
        base_ref[...] = jnp.zeros((1, E), jnp.float32)

    xu = lax.bitcast_convert_type(x_ref[...], jnp.int32)
    rnd = 0x7FFF + ((xu >> 16) & 1)
    xb16 = ((xu + rnd) >> 16) & 0xFFFF
    lo = xb16[:, : D_MODEL // 2]
    hi = xb16[:, D_MODEL // 2:]
    xb_ref[...] = (hi << 16) | lo

    logits = jnp.dot(x_ref[...], wg_ref[...], preferred_element_type=jnp.float32)
    iota = lax.broadcasted_iota(jnp.int32, logits.shape, 1)
    m1 = jnp.max(logits, axis=1, keepdims=True)
    i1 = jnp.min(jnp.where(logits == m1, iota, E), axis=1, keepdims=True)
    masked = jnp.where(iota == i1, -jnp.inf, logits)
    m2 = jnp.max(masked, axis=1, keepdims=True)
    i2 = jnp.min(jnp.where(masked == m2, iota, E), axis=1, keepdims=True)
    g1 = 1.0 / (1.0 + jnp.exp(m2 - m1))
    g1_ref[...] = jnp.broadcast_to(g1, (TB, 16))
    g2_ref[...] = jnp.broadcast_to(1.0 - g1, (TB, 16))

    oh1 = (iota == i1).astype(jnp.float32)
    oh2 = (iota == i2).astype(jnp.float32)
    ri = lax.broadcasted_iota(jnp.int32, (TB, TB), 0)
    ci = lax.broadcasted_iota(jnp.int32, (TB, TB), 1)
    ltri = (ri > ci).astype(jnp.float32)
    cum = jnp.dot(ltri, oh1 + oh2, preferred_element_type=jnp.float32)
    base = base_ref[...]
    r1 = jnp.sum(oh1 * (base + cum), axis=1, keepdims=True)
    r2 = jnp.sum(oh2 * (base + cum + oh1), axis=1, keepdims=True)
    sl = pl.ds(pid * TB, TB)
    i1s[sl, :] = i1
    i2s[sl, :] = i2
    r1s[sl, :] = r1
    r2s[sl, :] = r2
    newbase = base + jnp.sum(oh1 + oh2, axis=0, keepdims=True)
    base_ref[...] = newbase

    @pl.when(pid == NTB - 1)
    def _():
        counts = newbase
        nblk = jnp.floor((counts + (BLK - 1)) * (1.0 / BLK))
        ei = lax.broadcasted_iota(jnp.int32, (E, E), 0)
        ej = lax.broadcasted_iota(jnp.int32, (E, E), 1)
        incl = (ei <= ej).astype(jnp.float32)
        cum_incl = jnp.dot(nblk, incl, preferred_element_type=jnp.float32)
        blk_start = cum_incl - nblk
        row_off = blk_start * float(BLK)

        it = lax.broadcasted_iota(jnp.int32, (S, E), 1)
        sel1 = (it == i1s[...]).astype(jnp.float32)
        sel2 = (it == i2s[...]).astype(jnp.float32)
        d1 = jnp.sum(sel1 * row_off, axis=1, keepdims=True) + r1s[...]
        d2 = jnp.sum(sel2 * row_off, axis=1, keepdims=True) + r2s[...]
        d1_ref[...] = d1.astype(jnp.int32)
        d2_ref[...] = d2.astype(jnp.int32)

        bi = lax.broadcasted_iota(jnp.int32, (NBP, E), 0)
        be = lax.broadcasted_iota(jnp.int32, (NBP, E), 1)
        active = (bi >= blk_start.astype(jnp.int32)) & (nblk > 0.0)
        b2e_ref[...] = jnp.max(jnp.where(active, be, 0), axis=1, keepdims=True)
        nact_ref[...] = jnp.sum(nblk, axis=1, keepdims=True).astype(jnp.int32)


def _router(x, wg):
    outs = [
        jax.ShapeDtypeStruct((S, 16), jnp.float32),
        jax.ShapeDtypeStruct((S, 16), jnp.float32),
        jax.ShapeDtypeStruct((S, 1), jnp.int32),
        jax.ShapeDtypeStruct((S, 1), jnp.int32),
        jax.ShapeDtypeStruct((NBP, 1), jnp.int32),
        jax.ShapeDtypeStruct((1, 1), jnp.int32),
        jax.ShapeDtypeStruct((S, D_MODEL // 2), jnp.int32),
    ]
    tokspec = pl.BlockSpec((TB, 16), lambda i: (i, 0))
    whole = lambda i: (0, 0)
    return pl.pallas_call(
        _router_body,
        grid=(NTB,),
        in_specs=[
            pl.BlockSpec((TB, D_MODEL), lambda i: (i, 0)),
            pl.BlockSpec((D_MODEL, E), whole),
        ],
        out_specs=[tokspec, tokspec,
                   pl.BlockSpec((S, 1), whole), pl.BlockSpec((S, 1), whole),
                   pl.BlockSpec((NBP, 1), whole), pl.BlockSpec((1, 1), whole),
                   pl.BlockSpec((TB, D_MODEL // 2), lambda i: (i, 0))],
        out_shape=outs,
        scratch_shapes=[pltpu.VMEM((1, E), jnp.float32),
                        pltpu.VMEM((S, 1), jnp.int32),
                        pltpu.VMEM((S, 1), jnp.int32),
                        pltpu.VMEM((S, 1), jnp.float32),
                        pltpu.VMEM((S, 1), jnp.float32)],
    )(x, wg)


def _sc_dispatch(x2d, d1w, d2w):
    mesh = plsc.VectorSubcoreMesh(core_axis_name="c", subcore_axis_name="s")

    @functools.partial(
        pl.kernel,
        mesh=mesh,
        out_type=jax.ShapeDtypeStruct((P_PAD, D_MODEL // 2), jnp.int32),
        scratch_types=[
            pltpu.VMEM((TOP_K, _TPW), jnp.int32),
            pltpu.VMEM((_TPW, D_MODEL // 2), jnp.int32),
            pltpu.SemaphoreType.DMA,
        ],
    )
    def k(x_hbm, d1_hbm, d2_hbm, out_hbm, idx_v, buf, sem):
        wid = lax.axis_index("s") * 2 + lax.axis_index("c")
        pltpu.sync_copy(d1_hbm.at[wid], idx_v.at[0])
        pltpu.sync_copy(d2_hbm.at[wid], idx_v.at[1])
        pltpu.sync_copy(x_hbm.at[pl.ds(wid * _TPW, _TPW)], buf)
        ca = pltpu.async_copy(buf, out_hbm.at[idx_v.at[0]], sem)
        cb = pltpu.async_copy(buf, out_hbm.at[idx_v.at[1]], sem)
        ca.wait()
        cb.wait()

    return k(x2d, d1w, d2w)


def _ffn_body(b2e_ref, nact_ref, x_ref, w1_ref, b1_ref, w2_ref, b2_ref, y_ref):
    pid = pl.program_id(0)

    @pl.when(pid < nact_ref[0])
    def _():
        w = x_ref[...]
        lo = lax.bitcast_convert_type(w << 16, jnp.float32)
        hi = lax.bitcast_convert_type(w & jnp.int32(-65536), jnp.float32)
        x = jnp.concatenate([lo, hi], axis=1)
        h = jnp.dot(x, w1_ref[0], preferred_element_type=jnp.float32)
        h = jax.nn.gelu(h + b1_ref[0])
        y = jnp.dot(h, w2_ref[0], preferred_element_type=jnp.float32)
        y = y + b2_ref[0]
        yu = lax.bitcast_convert_type(y, jnp.int32)
        rnd = 0x7FFF + ((yu >> 16) & 1)
        y16 = ((yu + rnd) >> 16) & 0xFFFF
        ylo = y16[:, : D_MODEL // 2]
        yhi = y16[:, D_MODEL // 2:]
        y_ref[...] = (yhi << 16) | ylo


def _ffn(x_pad, w1, b1, w2, b2, b2e, nact):
    clamp = lambda i, na: jnp.where(i < na[0], i, na[0] - 1)
    grid_spec = pltpu.PrefetchScalarGridSpec(
        num_scalar_prefetch=2,
        grid=(NB,),
        in_specs=[
            pl.BlockSpec((BLK, D_MODEL // 2),
                         lambda i, b2e, na: (clamp(i, na), 0)),
            pl.BlockSpec((1, D_MODEL, D_FF), lambda i, b2e, na: (b2e[i], 0, 0)),
            pl.BlockSpec((1, 1, D_FF), lambda i, b2e, na: (b2e[i], 0, 0)),
            pl.BlockSpec((1, D_FF, D_MODEL), lambda i, b2e, na: (b2e[i], 0, 0)),
            pl.BlockSpec((1, 1, D_MODEL), lambda i, b2e, na: (b2e[i], 0, 0)),
        ],
        out_specs=pl.BlockSpec((BLK, D_MODEL // 2),
                               lambda i, b2e, na: (clamp(i, na), 0)),
    )
    return pl.pallas_call(
        _ffn_body,
        grid_spec=grid_spec,
        out_shape=jax.ShapeDtypeStruct((P_PAD, D_MODEL // 2), jnp.int32),
    )(b2e, nact, x_pad, w1, b1, w2, b2)


def _sc_combine(y_pad, d1w, d2w, g1w, g2w):
    mesh = plsc.VectorSubcoreMesh(core_axis_name="c", subcore_axis_name="s")

    @functools.partial(
        pl.kernel,
        mesh=mesh,
        out_type=jax.ShapeDtypeStruct((S, D_MODEL), jnp.float32),
        scratch_types=[
            pltpu.VMEM((_TPW,), jnp.int32),
            pltpu.VMEM((_TPW,), jnp.int32),
            pltpu.VMEM((_TPW, 16), jnp.float32),
            pltpu.VMEM((_TPW, 16), jnp.float32),
            pltpu.VMEM((2, _CCH, D_MODEL // 2), jnp.int32),
            pltpu.VMEM((2, _CCH, D_MODEL // 2), jnp.int32),
            pltpu.VMEM((2, _CCH, D_MODEL), jnp.float32),
            pltpu.SemaphoreType.DMA,
            pltpu.SemaphoreType.DMA,
            pltpu.SemaphoreType.DMA,
            pltpu.SemaphoreType.DMA,
            pltpu.SemaphoreType.DMA,
            pltpu.SemaphoreType.DMA,
        ],
    )
    def k(y_hbm, d1_hbm, d2_hbm, g1_hbm, g2_hbm, out_hbm,
          pa_v, pb_v, ga_v, gb_v, bufa, bufb, bufo,
          sga0, sga1, sgb0, sgb1, swr0, swr1):
        sga = [sga0, sga1]
        sgb = [sgb0, sgb1]
        swr = [swr0, swr1]
        wid = lax.axis_index("s") * 2 + lax.axis_index("c")
        pltpu.sync_copy(d1_hbm.at[wid], pa_v)
        pltpu.sync_copy(d2_hbm.at[wid], pb_v)
        pltpu.sync_copy(g1_hbm.at[wid], ga_v)
        pltpu.sync_copy(g2_hbm.at[wid], gb_v)

        def gathers(c, s):
            cpa = pltpu.async_copy(
                y_hbm.at[pa_v.at[pl.ds(c * _CCH, _CCH)]], bufa.at[s], sga[s])
            cpb = pltpu.async_copy(
                y_hbm.at[pb_v.at[pl.ds(c * _CCH, _CCH)]], bufb.at[s], sgb[s])
            return cpa, cpb

        pend = [gathers(0, 0), gathers(1, 1)]
        wr = [None, None]
        for c in range(_NCC):
            s = c % 2
            cpa, cpb = pend[s]
            cpa.wait()
            cpb.wait()
            if wr[s] is not None:
                wr[s].wait()

            def body(r, carry, c=c, s=s):
                tok = c * _CCH + r
                ga = ga_v[tok, :]
                gb = gb_v[tok, :]
                for q in range(D_MODEL // 32):
                    sl = pl.ds(q * 16, 16)
                    wa = bufa[s, r, sl]
                    wb = bufb[s, r, sl]
                    a_lo = lax.bitcast_convert_type(wa << 16, jnp.float32)
                    a_hi = lax.bitcast_convert_type(
                        wa & jnp.int32(-65536), jnp.float32)
                    b_lo = lax.bitcast_convert_type(wb << 16, jnp.float32)
                    b_hi = lax.bitcast_convert_type(
                        wb & jnp.int32(-65536), jnp.float32)
                    bufo[s, r, sl] = ga * a_lo + gb * b_lo
                    bufo[s, r, pl.ds(D_MODEL // 2 + q * 16, 16)] = (
                        ga * a_hi + gb * b_hi)
                return carry

            lax.fori_loop(0, _CCH, body, 0)
            if c + 2 < _NCC:
                pend[s] = gathers(c + 2, s)
            wr[s] = pltpu.async_copy(
                bufo.at[s], out_hbm.at[pl.ds(wid * _TPW + c * _CCH, _CCH)],
                swr[s])
        for s in range(2):
            if wr[s] is not None:
                wr[s].wait()

    return k(y_pad, d1w, d2w, g1w, g2w)


def kernel(hidden_states, Wg, W1, b1, W2, b2):
    x = hidden_states.reshape(S, D_MODEL)
    g1, g2, dest1, dest2, b2e, nact, xb = _router(x, Wg)

    d1w = dest1.reshape(_NW, _TPW)
    d2w = dest2.reshape(_NW, _TPW)
    x_pad = _sc_dispatch(xb, d1w, d2w)
    y_pad = _ffn(x_pad, W1, b1.reshape(E, 1, D_FF), W2,
                 b2.reshape(E, 1, D_MODEL), b2e.reshape(NBP), nact.reshape(1))

    out = _sc_combine(y_pad, d1w, d2w,
                      g1.reshape(_NW, _TPW, 16), g2.reshape(_NW, _TPW, 16))
    return out.reshape(hidden_states.shape)
